# baseline (speedup 1.0000x reference)
import functools

import jax
import jax.numpy as jnp
from jax import lax
from jax.experimental import pallas as pl
from jax.experimental.pallas import tpu as pltpu
from jax.experimental.pallas import tpu_sc as plsc

VOCAB = 1_000_000
EMB = 256
HID = 256
OUT = 18
BATCH = 16384

_info = plsc.get_sparse_core_info()
_NC, _NS = _info.num_cores, _info.num_subcores
_NW = _NC * _NS
_CHUNK = 128

_mesh = plsc.VectorSubcoreMesh(core_axis_name="c", subcore_axis_name="s")


def _make_gather(nrows):
    bpw = nrows // _NW
    nchunk = bpw // _CHUNK
    assert nchunk * _CHUNK == bpw and nchunk >= 1

    @functools.partial(
        pl.kernel,
        mesh=_mesh,
        out_type=jax.ShapeDtypeStruct((nrows, EMB), jnp.float32),
        scratch_types=[
            pltpu.VMEM((bpw,), jnp.int32),
            pltpu.VMEM((_CHUNK, EMB), jnp.float32),
            pltpu.VMEM((_CHUNK, EMB), jnp.float32),
            pltpu.SemaphoreType.DMA,
            pltpu.SemaphoreType.DMA,
            pltpu.SemaphoreType.DMA,
            pltpu.SemaphoreType.DMA,
        ],
    )
    def gather_sc(idx_hbm, table_hbm, out_hbm, idx_v, rows0, rows1,
                  gsem0, gsem1, ssem0, ssem1):
        wid = lax.axis_index("s") * _NC + lax.axis_index("c")
        base = wid * bpw
        bufs = (rows0, rows1)
        gsems = (gsem0, gsem1)
        ssems = (ssem0, ssem1)
        pltpu.sync_copy(idx_hbm.at[pl.ds(base, bpw)], idx_v)

        def gather(c):
            return pltpu.async_copy(
                table_hbm.at[idx_v.at[pl.ds(c * _CHUNK, _CHUNK)]],
                bufs[c % 2], gsems[c % 2])

        def store(c):
            return pltpu.async_copy(
                bufs[c % 2], out_hbm.at[pl.ds(base + c * _CHUNK, _CHUNK)],
                ssems[c % 2])

        g = [None] * nchunk
        s = [None] * nchunk
        g[0] = gather(0)
        if nchunk > 1:
            g[1] = gather(1)
        g[0].wait()
        s[0] = store(0)
        for c in range(1, nchunk):
            g[c].wait()
            s[c] = store(c)
            if c + 1 < nchunk:
                s[c - 1].wait()
                g[c + 1] = gather(c + 1)
        if nchunk > 1:
            s[nchunk - 2].wait()
        s[nchunk - 1].wait()

    return gather_sc


def _make_gather_bf16(nrows):
    bpw = nrows // _NW
    nchunk = bpw // _CHUNK
    assert nchunk * _CHUNK == bpw and nchunk >= 1

    @functools.partial(
        pl.kernel,
        mesh=_mesh,
        out_type=jax.ShapeDtypeStruct((nrows, EMB), jnp.bfloat16),
        scratch_types=[
            pltpu.VMEM((bpw,), jnp.int32),
            pltpu.VMEM((_CHUNK, EMB), jnp.float32),
            pltpu.VMEM((_CHUNK, EMB), jnp.float32),
            pltpu.VMEM((_CHUNK, EMB), jnp.bfloat16),
            pltpu.VMEM((_CHUNK, EMB), jnp.bfloat16),
            pltpu.SemaphoreType.DMA,
            pltpu.SemaphoreType.DMA,
            pltpu.SemaphoreType.DMA,
            pltpu.SemaphoreType.DMA,
        ],
    )
    def gather_sc(idx_hbm, table_hbm, out_hbm, idx_v, rows0, rows1, ob0, ob1,
                  gsem0, gsem1, ssem0, ssem1):
        wid = lax.axis_index("s") * _NC + lax.axis_index("c")
        base = wid * bpw
        bufs = (rows0, rows1)
        obufs = (ob0, ob1)
        gsems = (gsem0, gsem1)
        ssems = (ssem0, ssem1)
        pltpu.sync_copy(idx_hbm.at[pl.ds(base, bpw)], idx_v)

        def gather(c):
            return pltpu.async_copy(
                table_hbm.at[idx_v.at[pl.ds(c * _CHUNK, _CHUNK)]],
                bufs[c % 2], gsems[c % 2])

        def convert(c):
            rows = bufs[c % 2]
            ob = obufs[c % 2]

            def body(i, carry):
                col = pl.multiple_of(i * 16, 16)
                for r in range(_CHUNK):
                    ob[r, pl.ds(col, 16)] = (
                        rows[r, pl.ds(col, 16)].astype(jnp.bfloat16))
                return carry

            lax.fori_loop(0, EMB // 16, body, 0)

        def store(c):
            return pltpu.async_copy(
                obufs[c % 2], out_hbm.at[pl.ds(base + c * _CHUNK, _CHUNK)],
                ssems[c % 2])

        g = [None] * nchunk
        s = [None] * nchunk
        g[0] = gather(0)
        if nchunk > 1:
            g[1] = gather(1)
        g[0].wait()
        convert(0)
        s[0] = store(0)
        for c in range(1, nchunk):
            g[c].wait()
            convert(c)
            s[c] = store(c)
            if c + 1 < nchunk:
                s[c - 1].wait()
                g[c + 1] = gather(c + 1)
        if nchunk > 1:
            s[nchunk - 2].wait()
        s[nchunk - 1].wait()

    return gather_sc


_PERM = [32 * (j // 32) + (j % 2) * 16 + (j % 32) // 2 for j in range(EMB)]


def _mlp_body(x_ref, w2_ref, b2_ref, w3_ref, b3_ref, o_ref):
    h = lax.dot_general(
        x_ref[...].astype(jnp.float32), w2_ref[...],
        (((1,), (1,)), ((), ())),
        preferred_element_type=jnp.float32,
    ) + b2_ref[...]
    h = jnp.maximum(h, 0.0)
    o_ref[...] = lax.dot_general(
        h, w3_ref[...],
        (((1,), (1,)), ((), ())),
        preferred_element_type=jnp.float32,
    ) + b3_ref[...]


def _mlp(x, W2, b2r, W3, b3r, bs):
    nb = x.shape[0]
    return pl.pallas_call(
        _mlp_body,
        grid=(nb // bs,),
        in_specs=[
            pl.BlockSpec((bs, EMB), lambda i: (i, 0)),
            pl.BlockSpec((HID, EMB), lambda i: (0, 0)),
            pl.BlockSpec((1, HID), lambda i: (0, 0)),
            pl.BlockSpec((OUT, HID), lambda i: (0, 0)),
            pl.BlockSpec((1, OUT), lambda i: (0, 0)),
        ],
        out_specs=pl.BlockSpec((bs, OUT), lambda i: (i, 0)),
        out_shape=jax.ShapeDtypeStruct((nb, OUT), jnp.float32),
    )(x, W2, b2r, W3, b3r)


_NCHAIN = 2
_ROWS = BATCH // _NCHAIN
_gather = _make_gather_bf16(_ROWS)


def kernel(states, emb, W2, b2, W3, b3):
    idx = states.astype(jnp.int32).reshape(_NCHAIN, _ROWS)
    b2r = b2.reshape(1, HID)
    b3r = b3.reshape(1, OUT)
    outs = []
    for i in range(_NCHAIN):
        x = _gather(idx[i], emb)
        outs.append(_mlp(x, W2, b2r, W3, b3r, bs=2048))
    return jnp.concatenate(outs, axis=0)


# speedup vs baseline: 2.1097x; 2.1097x over previous
import functools

import jax
import jax.numpy as jnp
from jax import lax
from jax.experimental import pallas as pl
from jax.experimental.pallas import tpu as pltpu
from jax.experimental.pallas import tpu_sc as plsc

VOCAB = 1_000_000
EMB = 256
HID = 256
OUT = 18
BATCH = 16384

_info = plsc.get_sparse_core_info()
_NC, _NS = _info.num_cores, _info.num_subcores
_NW = _NC * _NS
_CHUNK = 128

_mesh = plsc.VectorSubcoreMesh(core_axis_name="c", subcore_axis_name="s")


def _make_gather(nrows):
    bpw = nrows // _NW
    nchunk = bpw // _CHUNK
    assert nchunk * _CHUNK == bpw and nchunk >= 1

    @functools.partial(
        pl.kernel,
        mesh=_mesh,
        out_type=jax.ShapeDtypeStruct((nrows, EMB), jnp.float32),
        scratch_types=[
            pltpu.VMEM((bpw,), jnp.int32),
            pltpu.VMEM((_CHUNK, EMB), jnp.float32),
            pltpu.VMEM((_CHUNK, EMB), jnp.float32),
            pltpu.SemaphoreType.DMA,
            pltpu.SemaphoreType.DMA,
            pltpu.SemaphoreType.DMA,
            pltpu.SemaphoreType.DMA,
        ],
    )
    def gather_sc(idx_hbm, table_hbm, out_hbm, idx_v, rows0, rows1,
                  gsem0, gsem1, ssem0, ssem1):
        wid = lax.axis_index("s") * _NC + lax.axis_index("c")
        base = wid * bpw
        bufs = (rows0, rows1)
        gsems = (gsem0, gsem1)
        ssems = (ssem0, ssem1)
        pltpu.sync_copy(idx_hbm.at[pl.ds(base, bpw)], idx_v)

        def gather(c):
            return pltpu.async_copy(
                table_hbm.at[idx_v.at[pl.ds(c * _CHUNK, _CHUNK)]],
                bufs[c % 2], gsems[c % 2])

        def store(c):
            return pltpu.async_copy(
                bufs[c % 2], out_hbm.at[pl.ds(base + c * _CHUNK, _CHUNK)],
                ssems[c % 2])

        g = [None] * nchunk
        s = [None] * nchunk
        g[0] = gather(0)
        if nchunk > 1:
            g[1] = gather(1)
        g[0].wait()
        s[0] = store(0)
        for c in range(1, nchunk):
            g[c].wait()
            s[c] = store(c)
            if c + 1 < nchunk:
                s[c - 1].wait()
                g[c + 1] = gather(c + 1)
        if nchunk > 1:
            s[nchunk - 2].wait()
        s[nchunk - 1].wait()

    return gather_sc


def _mlp_body(x_ref, w2_ref, b2_ref, w3_ref, b3_ref, o_ref):
    h = lax.dot_general(
        x_ref[...], w2_ref[...],
        (((1,), (1,)), ((), ())),
        preferred_element_type=jnp.float32,
    ) + b2_ref[...]
    h = jnp.maximum(h, 0.0)
    o_ref[...] = lax.dot_general(
        h, w3_ref[...],
        (((1,), (1,)), ((), ())),
        preferred_element_type=jnp.float32,
    ) + b3_ref[...]


def _mlp(x, W2, b2r, W3, b3r, bs):
    nb = x.shape[0]
    return pl.pallas_call(
        _mlp_body,
        grid=(nb // bs,),
        in_specs=[
            pl.BlockSpec((bs, EMB), lambda i: (i, 0)),
            pl.BlockSpec((HID, EMB), lambda i: (0, 0)),
            pl.BlockSpec((1, HID), lambda i: (0, 0)),
            pl.BlockSpec((OUT, HID), lambda i: (0, 0)),
            pl.BlockSpec((1, OUT), lambda i: (0, 0)),
        ],
        out_specs=pl.BlockSpec((bs, OUT), lambda i: (i, 0)),
        out_shape=jax.ShapeDtypeStruct((nb, OUT), jnp.float32),
    )(x, W2, b2r, W3, b3r)


_NCHAIN = 2
_ROWS = BATCH // _NCHAIN
_gather = _make_gather(_ROWS)


def kernel(states, emb, W2, b2, W3, b3):
    idx = states.astype(jnp.int32).reshape(_NCHAIN, _ROWS)
    b2r = b2.reshape(1, HID)
    b3r = b3.reshape(1, OUT)
    xs = [_gather(idx[i], emb) for i in range(_NCHAIN)]
    outs = [_mlp(x, W2, b2r, W3, b3r, bs=2048) for x in xs]
    return jnp.concatenate(outs, axis=0)
